# baseline (speedup 1.0000x reference)
import functools

import jax
import jax.numpy as jnp
from jax import lax
from jax.experimental import pallas as pl
from jax.experimental.pallas import tpu as pltpu
from jax.experimental.pallas import tpu_sc as plsc

CONFIDENCE = 0.8
SMOOTHING = 1.0 - CONFIDENCE

ROW_TILE = 512
V_TILE = 6400

_NC, _NS, _NL = 2, 16, 16
_NW = _NC * _NS


def _fill_kernel(out_ref, *, base):
    out_ref[...] = jnp.full(out_ref.shape, base, out_ref.dtype)


def _sc_scatter_body(out_hbm, ix_hbm, idx_v, conf_v, sem, *, v, rpw):
    wid = lax.axis_index("s") * _NC + lax.axis_index("c")
    row0 = wid * rpw
    pltpu.sync_copy(ix_hbm.at[pl.ds(row0, rpw)], idx_v)
    for j in range(rpw // _NL):
        rows = lax.iota(jnp.int32, _NL) + (row0 + j * _NL)
        flat = rows * v + idx_v[pl.ds(j * _NL, _NL)]
        idx_v[pl.ds(j * _NL, _NL)] = flat
        conf_v[pl.ds(j * _NL, _NL)] = jnp.full((_NL,), CONFIDENCE, jnp.float32)
    pltpu.async_copy(conf_v, out_hbm.at[idx_v], sem).wait()


def kernel(prediction, ix):
    B, S, V = prediction.shape
    R = B * S
    base = SMOOTHING / (V - 1)
    rpw = R // _NW

    filled = pl.pallas_call(
        functools.partial(_fill_kernel, base=base),
        grid=(R // ROW_TILE, V // V_TILE),
        out_specs=pl.BlockSpec((ROW_TILE, V_TILE), lambda i, j: (i, j)),
        out_shape=jax.ShapeDtypeStruct((R, V), prediction.dtype),
    )()

    out_ref = jax.new_ref(filled.reshape(R * V))
    scatter = pl.kernel(
        functools.partial(_sc_scatter_body, v=V, rpw=rpw),
        out_type=(),
        mesh=plsc.VectorSubcoreMesh(core_axis_name="c", subcore_axis_name="s"),
        scratch_types=[
            pltpu.VMEM((rpw,), jnp.int32),
            pltpu.VMEM((rpw,), jnp.float32),
            pltpu.SemaphoreType.DMA,
        ],
    )
    scatter(out_ref, ix.reshape(R))
    return out_ref[...].reshape(B, S, V)


# speedup vs baseline: 1.6138x; 1.6138x over previous
import functools

import jax
import jax.numpy as jnp
from jax import lax
from jax.experimental import pallas as pl
from jax.experimental.pallas import tpu as pltpu
from jax.experimental.pallas import tpu_sc as plsc

CONFIDENCE = 0.8
SMOOTHING = 1.0 - CONFIDENCE

_NC, _NS, _NL = 2, 16, 16
_NW = _NC * _NS

CHUNK = 2_048_000
FILL_STORE = 128_000


def _tc_fill_body(out_hbm, buf, sem, *, base, n_chunks):
    for i in range(CHUNK // FILL_STORE):
        buf[pl.ds(i * FILL_STORE, FILL_STORE)] = jnp.full(
            (FILL_STORE,), base, jnp.float32
        )
    copies = [
        pltpu.make_async_copy(buf, out_hbm.at[pl.ds(c * CHUNK, CHUNK)], sem)
        for c in range(n_chunks)
    ]
    for c in copies:
        c.start()
    for c in copies:
        c.wait()


def _sc_scatter_body(out_hbm, ix_hbm, idx_v, conf_v, sem, *, v, rpw):
    wid = lax.axis_index("s") * _NC + lax.axis_index("c")
    row0 = wid * rpw
    pltpu.sync_copy(ix_hbm.at[pl.ds(row0, rpw)], idx_v)
    for j in range(rpw // _NL):
        rows = lax.iota(jnp.int32, _NL) + (row0 + j * _NL)
        flat = rows * v + idx_v[pl.ds(j * _NL, _NL)]
        idx_v[pl.ds(j * _NL, _NL)] = flat
        conf_v[pl.ds(j * _NL, _NL)] = jnp.full((_NL,), CONFIDENCE, jnp.float32)
    pltpu.async_copy(conf_v, out_hbm.at[idx_v], sem).wait()


def kernel(prediction, ix):
    B, S, V = prediction.shape
    R = B * S
    flat = R * V
    base = SMOOTHING / (V - 1)
    rpw = R // _NW
    n_chunks = flat // CHUNK

    out_ref = jax.empty_ref(jax.ShapeDtypeStruct((flat,), prediction.dtype))

    fill = pl.kernel(
        functools.partial(_tc_fill_body, base=base, n_chunks=n_chunks),
        out_type=(),
        mesh=pltpu.create_tensorcore_mesh("x", num_cores=1),
        scratch_types=[
            pltpu.VMEM((CHUNK,), jnp.float32),
            pltpu.SemaphoreType.DMA,
        ],
    )
    fill(out_ref)

    scatter = pl.kernel(
        functools.partial(_sc_scatter_body, v=V, rpw=rpw),
        out_type=(),
        mesh=plsc.VectorSubcoreMesh(
            core_axis_name="c",
            subcore_axis_name="s",
            num_cores=_NC,
            num_subcores=_NS,
        ),
        scratch_types=[
            pltpu.VMEM((rpw,), jnp.int32),
            pltpu.VMEM((rpw,), jnp.float32),
            pltpu.SemaphoreType.DMA,
        ],
    )
    scatter(out_ref, ix.reshape(R))
    return jax.freeze(out_ref).reshape(B, S, V)


# speedup vs baseline: 1.6142x; 1.0002x over previous
import functools

import jax
import jax.numpy as jnp
from jax import lax
from jax.experimental import pallas as pl
from jax.experimental.pallas import tpu as pltpu
from jax.experimental.pallas import tpu_sc as plsc

CONFIDENCE = 0.8
SMOOTHING = 1.0 - CONFIDENCE

_NC, _NS, _NL = 2, 16, 16
_NW = _NC * _NS

CHUNK = 1_024_000
FILL_STORE = 128_000
NQ = 8


def _tc_fill_body(out_hbm, bufs, sems, *, base, n_chunks):
    nq = len(sems)
    for buf in bufs:
        for i in range(CHUNK // FILL_STORE):
            buf[pl.ds(i * FILL_STORE, FILL_STORE)] = jnp.full(
                (FILL_STORE,), base, jnp.float32
            )
    copies = [
        pltpu.make_async_copy(
            bufs[c % nq], out_hbm.at[pl.ds(c * CHUNK, CHUNK)], sems[c % nq]
        )
        for c in range(n_chunks)
    ]
    for c in copies:
        c.start()
    for c in copies:
        c.wait()


def _sc_scatter_body(out_hbm, ix_hbm, idx_v, conf_v, sem, *, v, rpw):
    wid = lax.axis_index("s") * _NC + lax.axis_index("c")
    row0 = wid * rpw
    pltpu.sync_copy(ix_hbm.at[pl.ds(row0, rpw)], idx_v)
    for j in range(rpw // _NL):
        rows = lax.iota(jnp.int32, _NL) + (row0 + j * _NL)
        flat = rows * v + idx_v[pl.ds(j * _NL, _NL)]
        idx_v[pl.ds(j * _NL, _NL)] = flat
        conf_v[pl.ds(j * _NL, _NL)] = jnp.full((_NL,), CONFIDENCE, jnp.float32)
    pltpu.async_copy(conf_v, out_hbm.at[idx_v], sem).wait()


def kernel(prediction, ix):
    B, S, V = prediction.shape
    R = B * S
    flat = R * V
    base = SMOOTHING / (V - 1)
    rpw = R // _NW
    n_chunks = flat // CHUNK

    out_ref = jax.empty_ref(jax.ShapeDtypeStruct((flat,), prediction.dtype))

    fill = pl.kernel(
        functools.partial(_tc_fill_body, base=base, n_chunks=n_chunks),
        out_type=(),
        mesh=pltpu.create_tensorcore_mesh("x", num_cores=1),
        scratch_types=[
            [pltpu.VMEM((CHUNK,), jnp.float32) for _ in range(NQ)],
            [pltpu.SemaphoreType.DMA for _ in range(NQ)],
        ],
    )
    fill(out_ref)

    scatter = pl.kernel(
        functools.partial(_sc_scatter_body, v=V, rpw=rpw),
        out_type=(),
        mesh=plsc.VectorSubcoreMesh(
            core_axis_name="c",
            subcore_axis_name="s",
            num_cores=_NC,
            num_subcores=_NS,
        ),
        scratch_types=[
            pltpu.VMEM((rpw,), jnp.int32),
            pltpu.VMEM((rpw,), jnp.float32),
            pltpu.SemaphoreType.DMA,
        ],
    )
    scatter(out_ref, ix.reshape(R))
    return jax.freeze(out_ref).reshape(B, S, V)
